# Initial kernel scaffold; baseline (speedup 1.0000x reference)
import functools

import jax
import jax.numpy as jnp
from jax import lax
from jax.experimental import pallas as pl
from jax.experimental.pallas import tpu as pltpu
from jax.experimental.pallas import tpu_sc as plsc

B = 16384
D = 64
NEG = 20
R = NEG + 1
NC, NS, L = 2, 16, 16
NW = NC * NS
BPW = B // NW
CB = 32
NCHUNK = BPW // CB
ROWS = CB * R
IW = 112
NIDX = ROWS // IW


def _sc_partial_dots(cent_idx, ctx_idx, U, V):
    mesh = plsc.VectorSubcoreMesh(
        core_axis_name="c", subcore_axis_name="s", num_cores=NC, num_subcores=NS
    )

    @functools.partial(
        pl.kernel,
        out_type=jax.ShapeDtypeStruct((B * R, L), jnp.float32),
        mesh=mesh,
        scratch_types=[
            pltpu.VMEM((1, CB), jnp.int32),
            pltpu.VMEM((NIDX, IW), jnp.int32),
            pltpu.VMEM((CB, D), jnp.float32),
            pltpu.VMEM((ROWS, D), jnp.float32),
            pltpu.VMEM((ROWS, L), jnp.float32),
            pltpu.SemaphoreType.DMA,
        ],
    )
    def k(cent_hbm, ctx_hbm, u_hbm, v_hbm, out_hbm, idx_c, idx_v, cent_rows,
          ctx_rows, part, sem):
        wid = lax.axis_index("s") * NC + lax.axis_index("c")

        def chunk_body(i, carry):
            chunk = wid * NCHUNK + i
            pltpu.sync_copy(cent_hbm.at[pl.ds(chunk, 1)], idx_c)
            pltpu.sync_copy(ctx_hbm.at[pl.ds(chunk * NIDX, NIDX)], idx_v)
            copies = [
                pltpu.async_copy(
                    v_hbm.at[idx_v.at[j]],
                    ctx_rows.at[pl.ds(j * IW, IW)],
                    sem,
                )
                for j in range(NIDX)
            ]
            copies.append(pltpu.async_copy(u_hbm.at[idx_c.at[0]], cent_rows, sem))
            for c in copies:
                c.wait()

            def b_body(b, carry2):
                c0 = cent_rows[b, pl.ds(0 * L, L)]
                c1 = cent_rows[b, pl.ds(1 * L, L)]
                c2 = cent_rows[b, pl.ds(2 * L, L)]
                c3 = cent_rows[b, pl.ds(3 * L, L)]
                for r in range(R):
                    row = b * R + r
                    acc = ctx_rows[row, pl.ds(0 * L, L)] * c0
                    acc = acc + ctx_rows[row, pl.ds(1 * L, L)] * c1
                    acc = acc + ctx_rows[row, pl.ds(2 * L, L)] * c2
                    acc = acc + ctx_rows[row, pl.ds(3 * L, L)] * c3
                    part[row, :] = acc
                return carry2

            lax.fori_loop(0, CB, b_body, 0)
            pltpu.sync_copy(part, out_hbm.at[pl.ds(chunk * ROWS, ROWS)])
            return carry

        lax.fori_loop(0, NCHUNK, chunk_body, 0)

    return k(cent_idx, ctx_idx, U, V)


_TC_GRID = 8
_TC_ROWS = B * R // 8 // _TC_GRID


def _tc_loss_body(pp_ref, out_ref):
    i = pl.program_id(0)
    x = pp_ref[...]
    lane = lax.broadcasted_iota(jnp.int32, (128, 8), 0)
    grp = lax.broadcasted_iota(jnp.int32, (128, 8), 1)
    onehot = (lane // L == grp).astype(jnp.float32)
    logits = jnp.dot(x, onehot, preferred_element_type=jnp.float32)
    p0 = lax.broadcasted_iota(jnp.int32, (_TC_ROWS, 8), 0) * 8
    p1 = lax.broadcasted_iota(jnp.int32, (_TC_ROWS, 8), 1)
    p = i * (_TC_ROWS * 8) + p0 + p1
    sign = jnp.where(p % R == 0, 1.0, -1.0)
    z = sign * logits
    contrib = jnp.minimum(z, 0.0) - jnp.log1p(jnp.exp(-jnp.abs(z)))

    @pl.when(i == 0)
    def _():
        out_ref[0, 0] = 0.0

    out_ref[0, 0] += -jnp.sum(contrib)


def _tc_loss(pp):
    pp2 = pp.reshape(B * R // 8, 128)
    return pl.pallas_call(
        _tc_loss_body,
        out_shape=jax.ShapeDtypeStruct((1, 1), jnp.float32),
        grid=(_TC_GRID,),
        in_specs=[pl.BlockSpec((_TC_ROWS, 128), lambda i: (i, 0))],
        out_specs=pl.BlockSpec((1, 1), lambda i: (0, 0)),
    )(pp2)


def kernel(cent_word, pos_word, neg_word, U, V):
    cent_idx = cent_word.astype(jnp.int32).reshape(B // CB, CB)
    ctx = jnp.concatenate(
        [pos_word.astype(jnp.int32), neg_word.astype(jnp.int32)], axis=1
    )
    ctx_idx = ctx.reshape(B * R // IW, IW)
    pp = _sc_partial_dots(cent_idx, ctx_idx, U, V)
    return _tc_loss(pp).reshape(())


# speedup vs baseline: 4.9013x; 4.9013x over previous
import functools

import jax
import jax.numpy as jnp
from jax import lax
from jax.experimental import pallas as pl
from jax.experimental.pallas import tpu as pltpu
from jax.experimental.pallas import tpu_sc as plsc

B = 16384
D = 64
NEG = 20
R = NEG + 1
NC, NS, L = 2, 16, 16
NW = NC * NS
BPW = B // NW
CB = 32
NCHUNK = BPW // CB
ROWS = CB * R
IW = 84
NIDX = ROWS // IW
IPW = NCHUNK * NIDX


def _sc_partial_dots(cent_idx, ctx_idx, U, V):
    mesh = plsc.VectorSubcoreMesh(
        core_axis_name="c", subcore_axis_name="s", num_cores=NC, num_subcores=NS
    )

    @functools.partial(
        pl.kernel,
        out_type=jax.ShapeDtypeStruct((B * R * L,), jnp.float32),
        mesh=mesh,
        compiler_params=pltpu.CompilerParams(use_tc_tiling_on_sc=False),
        scratch_types=[
            pltpu.VMEM((NCHUNK, CB), jnp.int32),
            pltpu.VMEM((IPW, IW), jnp.int32),
            pltpu.VMEM((CB, D), jnp.float32),
            pltpu.VMEM((ROWS, D), jnp.float32),
            pltpu.VMEM((ROWS * L,), jnp.float32),
            pltpu.SemaphoreType.DMA,
        ],
    )
    def k(cent_hbm, ctx_hbm, u_hbm, v_hbm, out_hbm, idx_c, idx_v, cent_rows,
          ctx_rows, part, sem):
        wid = lax.axis_index("s") * NC + lax.axis_index("c")
        pltpu.sync_copy(cent_hbm.at[pl.ds(wid * NCHUNK, NCHUNK)], idx_c)
        pltpu.sync_copy(ctx_hbm.at[pl.ds(wid * IPW, IPW)], idx_v)

        def chunk_body(i, carry):
            chunk = wid * NCHUNK + i
            copies = [
                pltpu.async_copy(
                    v_hbm.at[idx_v.at[i * NIDX + j]],
                    ctx_rows.at[pl.ds(j * IW, IW)],
                    sem,
                )
                for j in range(NIDX)
            ]
            copies.append(pltpu.async_copy(u_hbm.at[idx_c.at[i]], cent_rows, sem))
            for c in copies:
                c.wait()

            def b_body(b, carry2):
                c0 = cent_rows[b, pl.ds(0 * L, L)]
                c1 = cent_rows[b, pl.ds(1 * L, L)]
                c2 = cent_rows[b, pl.ds(2 * L, L)]
                c3 = cent_rows[b, pl.ds(3 * L, L)]
                for r in range(R):
                    row = b * R + r
                    acc = ctx_rows[row, pl.ds(0 * L, L)] * c0
                    acc = acc + ctx_rows[row, pl.ds(1 * L, L)] * c1
                    acc = acc + ctx_rows[row, pl.ds(2 * L, L)] * c2
                    acc = acc + ctx_rows[row, pl.ds(3 * L, L)] * c3
                    part[pl.ds(row * L, L)] = acc
                return carry2

            lax.fori_loop(0, CB, b_body, 0)
            pltpu.sync_copy(part, out_hbm.at[pl.ds(chunk * ROWS * L, ROWS * L)])
            return carry

        lax.fori_loop(0, NCHUNK, chunk_body, 0)

    return k(cent_idx, ctx_idx, U, V)


_TC_GRID = 8
_TC_ROWS = B * R * L // 128 // _TC_GRID


def _tc_loss_body(pp_ref, out_ref):
    i = pl.program_id(0)
    x = pp_ref[...]
    lane = lax.broadcasted_iota(jnp.int32, (128, 8), 0)
    grp = lax.broadcasted_iota(jnp.int32, (128, 8), 1)
    onehot = (lane // L == grp).astype(jnp.float32)
    logits = jnp.dot(x, onehot, preferred_element_type=jnp.float32)
    p0 = lax.broadcasted_iota(jnp.int32, (_TC_ROWS, 8), 0) * 8
    p1 = lax.broadcasted_iota(jnp.int32, (_TC_ROWS, 8), 1)
    p = i * (_TC_ROWS * 8) + p0 + p1
    sign = jnp.where(p % R == 0, 1.0, -1.0)
    z = sign * logits
    contrib = jnp.minimum(z, 0.0) - jnp.log1p(jnp.exp(-jnp.abs(z)))

    @pl.when(i == 0)
    def _():
        out_ref[...] = jnp.zeros_like(out_ref)

    out_ref[...] = out_ref[...] - jnp.sum(contrib)


def _tc_loss(pp):
    pp2 = pp.reshape(B * R * L // 128, 128)
    return pl.pallas_call(
        _tc_loss_body,
        out_shape=jax.ShapeDtypeStruct((1, 1), jnp.float32),
        grid=(_TC_GRID,),
        in_specs=[pl.BlockSpec((_TC_ROWS, 128), lambda i: (i, 0))],
        out_specs=pl.BlockSpec((1, 1), lambda i: (0, 0)),
    )(pp2)


def kernel(cent_word, pos_word, neg_word, U, V):
    cent_idx = cent_word.astype(jnp.int32).reshape(B // CB, CB)
    ctx = jnp.concatenate(
        [pos_word.astype(jnp.int32), neg_word.astype(jnp.int32)], axis=1
    )
    ctx_idx = ctx.reshape(B * R // IW, IW)
    pp = _sc_partial_dots(cent_idx, ctx_idx, U, V)
    return _tc_loss(pp).reshape(())


# speedup vs baseline: 4.9486x; 1.0096x over previous
import functools

import jax
import jax.numpy as jnp
from jax import lax
from jax.experimental import pallas as pl
from jax.experimental.pallas import tpu as pltpu
from jax.experimental.pallas import tpu_sc as plsc

B = 16384
D = 64
NEG = 20
NC, NS, L = 2, 16, 16
NW = NC * NS
BPW = B // NW
CB = 32
NCHUNK = BPW // CB
NROWS = CB * NEG
NIW = 128
NNI = NROWS // NIW
POS_SZ = B * L


def _sc_partial_dots(cent_idx, pos_idx, neg_idx, U, V):
    mesh = plsc.VectorSubcoreMesh(
        core_axis_name="c", subcore_axis_name="s", num_cores=NC, num_subcores=NS
    )

    @functools.partial(
        pl.kernel,
        out_type=jax.ShapeDtypeStruct((B * (NEG + 1) * L,), jnp.float32),
        mesh=mesh,
        compiler_params=pltpu.CompilerParams(use_tc_tiling_on_sc=False),
        scratch_types=[
            pltpu.VMEM((NCHUNK, CB), jnp.int32),
            pltpu.VMEM((NCHUNK, CB), jnp.int32),
            pltpu.VMEM((NCHUNK * NNI, NIW), jnp.int32),
            pltpu.VMEM((CB, D), jnp.float32),
            pltpu.VMEM((CB, D), jnp.float32),
            pltpu.VMEM((NROWS, D), jnp.float32),
            pltpu.VMEM((CB * L,), jnp.float32),
            pltpu.VMEM((NROWS * L,), jnp.float32),
            pltpu.SemaphoreType.DMA,
        ],
    )
    def k(cent_hbm, pos_hbm, neg_hbm, u_hbm, v_hbm, out_hbm, idx_c, idx_p,
          idx_n, cent_rows, pos_rows, neg_rows, part_p, part_n, sem):
        wid = lax.axis_index("s") * NC + lax.axis_index("c")
        pltpu.sync_copy(cent_hbm.at[pl.ds(wid * NCHUNK, NCHUNK)], idx_c)
        pltpu.sync_copy(pos_hbm.at[pl.ds(wid * NCHUNK, NCHUNK)], idx_p)
        pltpu.sync_copy(neg_hbm.at[pl.ds(wid * NCHUNK * NNI, NCHUNK * NNI)], idx_n)

        def chunk_body(i, carry):
            chunk = wid * NCHUNK + i
            copies = [
                pltpu.async_copy(
                    v_hbm.at[idx_n.at[i * NNI + j]],
                    neg_rows.at[pl.ds(j * NIW, NIW)],
                    sem,
                )
                for j in range(NNI)
            ]
            copies.append(pltpu.async_copy(v_hbm.at[idx_p.at[i]], pos_rows, sem))
            copies.append(pltpu.async_copy(u_hbm.at[idx_c.at[i]], cent_rows, sem))
            for c in copies:
                c.wait()

            def b_body(b, carry2):
                c0 = cent_rows[b, pl.ds(0 * L, L)]
                c1 = cent_rows[b, pl.ds(1 * L, L)]
                c2 = cent_rows[b, pl.ds(2 * L, L)]
                c3 = cent_rows[b, pl.ds(3 * L, L)]
                acc = pos_rows[b, pl.ds(0 * L, L)] * c0
                acc = acc + pos_rows[b, pl.ds(1 * L, L)] * c1
                acc = acc + pos_rows[b, pl.ds(2 * L, L)] * c2
                acc = acc + pos_rows[b, pl.ds(3 * L, L)] * c3
                part_p[pl.ds(b * L, L)] = acc
                for r in range(NEG):
                    row = b * NEG + r
                    acc = neg_rows[row, pl.ds(0 * L, L)] * c0
                    acc = acc + neg_rows[row, pl.ds(1 * L, L)] * c1
                    acc = acc + neg_rows[row, pl.ds(2 * L, L)] * c2
                    acc = acc + neg_rows[row, pl.ds(3 * L, L)] * c3
                    part_n[pl.ds(row * L, L)] = acc
                return carry2

            lax.fori_loop(0, CB, b_body, 0)
            pltpu.sync_copy(part_p, out_hbm.at[pl.ds(chunk * CB * L, CB * L)])
            pltpu.sync_copy(
                part_n, out_hbm.at[pl.ds(POS_SZ + chunk * NROWS * L, NROWS * L)]
            )
            return carry

        lax.fori_loop(0, NCHUNK, chunk_body, 0)

    return k(cent_idx, pos_idx, neg_idx, U, V)


_TC_GRID = 8
_TC_ROWS = B * (NEG + 1) * L // 128 // _TC_GRID
_POS_ROWS = POS_SZ // 128


def _tc_loss_body(pp_ref, out_ref):
    i = pl.program_id(0)
    x = pp_ref[...]
    lane = lax.broadcasted_iota(jnp.int32, (128, 8), 0)
    grp = lax.broadcasted_iota(jnp.int32, (128, 8), 1)
    onehot = (lane // L == grp).astype(jnp.float32)
    logits = jnp.dot(x, onehot, preferred_element_type=jnp.float32)
    row = i * _TC_ROWS + lax.broadcasted_iota(jnp.int32, (_TC_ROWS, 8), 0)
    sign = jnp.where(row < _POS_ROWS, 1.0, -1.0)
    z = sign * logits
    contrib = jnp.minimum(z, 0.0) - jnp.log1p(jnp.exp(-jnp.abs(z)))

    @pl.when(i == 0)
    def _():
        out_ref[...] = jnp.zeros_like(out_ref)

    out_ref[...] = out_ref[...] - jnp.sum(contrib)


def _tc_loss(pp):
    pp2 = pp.reshape(B * (NEG + 1) * L // 128, 128)
    return pl.pallas_call(
        _tc_loss_body,
        out_shape=jax.ShapeDtypeStruct((1, 1), jnp.float32),
        grid=(_TC_GRID,),
        in_specs=[pl.BlockSpec((_TC_ROWS, 128), lambda i: (i, 0))],
        out_specs=pl.BlockSpec((1, 1), lambda i: (0, 0)),
    )(pp2)


def kernel(cent_word, pos_word, neg_word, U, V):
    cent_idx = cent_word.astype(jnp.int32).reshape(B // CB, CB)
    pos_idx = pos_word.astype(jnp.int32).reshape(B // CB, CB)
    neg_idx = neg_word.astype(jnp.int32).reshape(B * NEG // NIW, NIW)
    pp = _sc_partial_dots(cent_idx, pos_idx, neg_idx, U, V)
    return _tc_loss(pp).reshape(())


# speedup vs baseline: 5.0602x; 1.0226x over previous
import functools

import jax
import jax.numpy as jnp
from jax import lax
from jax.experimental import pallas as pl
from jax.experimental.pallas import tpu as pltpu
from jax.experimental.pallas import tpu_sc as plsc

B = 16384
D = 64
NEG = 20
NC, NS, L = 2, 16, 16
NW = NC * NS
BPW = B // NW
CB = 32
NCHUNK = BPW // CB
NROWS = CB * NEG
NIW = 128
NNI = NROWS // NIW
POS_SZ = B * L


def _sc_partial_dots(cent_idx, pos_idx, neg_idx, U, V):
    mesh = plsc.VectorSubcoreMesh(
        core_axis_name="c", subcore_axis_name="s", num_cores=NC, num_subcores=NS
    )

    @functools.partial(
        pl.kernel,
        out_type=jax.ShapeDtypeStruct((B * (NEG + 1) * L,), jnp.float32),
        mesh=mesh,
        compiler_params=pltpu.CompilerParams(use_tc_tiling_on_sc=False),
        scratch_types=[
            pltpu.VMEM((NCHUNK, CB), jnp.int32),
            pltpu.VMEM((NCHUNK, CB), jnp.int32),
            pltpu.VMEM((NCHUNK * NNI, NIW), jnp.int32),
            pltpu.VMEM((2, CB, D), jnp.float32),
            pltpu.VMEM((2, CB, D), jnp.float32),
            pltpu.VMEM((2, NROWS, D), jnp.float32),
            pltpu.VMEM((2, CB * L), jnp.float32),
            pltpu.VMEM((2, NROWS * L), jnp.float32),
            pltpu.SemaphoreType.DMA,
            pltpu.SemaphoreType.DMA,
            pltpu.SemaphoreType.DMA,
        ],
    )
    def k(cent_hbm, pos_hbm, neg_hbm, u_hbm, v_hbm, out_hbm, idx_c, idx_p,
          idx_n, cent_rows, pos_rows, neg_rows, part_p, part_n, sem0, sem1,
          wsem):
        wid = lax.axis_index("s") * NC + lax.axis_index("c")
        pltpu.sync_copy(cent_hbm.at[pl.ds(wid * NCHUNK, NCHUNK)], idx_c)
        pltpu.sync_copy(pos_hbm.at[pl.ds(wid * NCHUNK, NCHUNK)], idx_p)
        pltpu.sync_copy(neg_hbm.at[pl.ds(wid * NCHUNK * NNI, NCHUNK * NNI)], idx_n)
        sems = (sem0, sem1)

        def fire(i, s):
            sem = sems[s]
            copies = [
                pltpu.async_copy(
                    v_hbm.at[idx_n.at[i * NNI + j]],
                    neg_rows.at[s].at[pl.ds(j * NIW, NIW)],
                    sem,
                )
                for j in range(NNI)
            ]
            copies.append(
                pltpu.async_copy(v_hbm.at[idx_p.at[i]], pos_rows.at[s], sem))
            copies.append(
                pltpu.async_copy(u_hbm.at[idx_c.at[i]], cent_rows.at[s], sem))
            return copies

        pending = fire(0, 0)
        wb = []
        for i in range(NCHUNK):
            s = i % 2
            chunk = wid * NCHUNK + i
            for c in pending:
                c.wait()
            if i + 1 < NCHUNK:
                pending = fire(i + 1, (i + 1) % 2)
            for c in wb:
                c.wait()
            wb = []
            cr, pr, nr = cent_rows.at[s], pos_rows.at[s], neg_rows.at[s]
            pp_, pn_ = part_p.at[s], part_n.at[s]

            def b_body(b, carry2, cr=cr, pr=pr, nr=nr, pp_=pp_, pn_=pn_):
                c0 = cr[b, pl.ds(0 * L, L)]
                c1 = cr[b, pl.ds(1 * L, L)]
                c2 = cr[b, pl.ds(2 * L, L)]
                c3 = cr[b, pl.ds(3 * L, L)]
                acc = pr[b, pl.ds(0 * L, L)] * c0
                acc = acc + pr[b, pl.ds(1 * L, L)] * c1
                acc = acc + pr[b, pl.ds(2 * L, L)] * c2
                acc = acc + pr[b, pl.ds(3 * L, L)] * c3
                pp_[pl.ds(b * L, L)] = acc
                for r in range(NEG):
                    row = b * NEG + r
                    acc = nr[row, pl.ds(0 * L, L)] * c0
                    acc = acc + nr[row, pl.ds(1 * L, L)] * c1
                    acc = acc + nr[row, pl.ds(2 * L, L)] * c2
                    acc = acc + nr[row, pl.ds(3 * L, L)] * c3
                    pn_[pl.ds(row * L, L)] = acc
                return carry2

            lax.fori_loop(0, CB, b_body, 0)
            wb = [
                pltpu.async_copy(
                    pp_, out_hbm.at[pl.ds(chunk * CB * L, CB * L)], wsem),
                pltpu.async_copy(
                    pn_,
                    out_hbm.at[pl.ds(POS_SZ + chunk * NROWS * L, NROWS * L)],
                    wsem,
                ),
            ]
        for c in wb:
            c.wait()

    return k(cent_idx, pos_idx, neg_idx, U, V)


_TC_GRID = 8
_TC_ROWS = B * (NEG + 1) * L // 128 // _TC_GRID
_POS_ROWS = POS_SZ // 128


def _tc_loss_body(pp_ref, out_ref):
    i = pl.program_id(0)
    x = pp_ref[...]
    lane = lax.broadcasted_iota(jnp.int32, (128, 8), 0)
    grp = lax.broadcasted_iota(jnp.int32, (128, 8), 1)
    onehot = (lane // L == grp).astype(jnp.float32)
    logits = jnp.dot(x, onehot, preferred_element_type=jnp.float32)
    row = i * _TC_ROWS + lax.broadcasted_iota(jnp.int32, (_TC_ROWS, 8), 0)
    sign = jnp.where(row < _POS_ROWS, 1.0, -1.0)
    z = sign * logits
    contrib = jnp.minimum(z, 0.0) - jnp.log1p(jnp.exp(-jnp.abs(z)))

    @pl.when(i == 0)
    def _():
        out_ref[...] = jnp.zeros_like(out_ref)

    out_ref[...] = out_ref[...] - jnp.sum(contrib)


def _tc_loss(pp):
    pp2 = pp.reshape(B * (NEG + 1) * L // 128, 128)
    return pl.pallas_call(
        _tc_loss_body,
        out_shape=jax.ShapeDtypeStruct((1, 1), jnp.float32),
        grid=(_TC_GRID,),
        in_specs=[pl.BlockSpec((_TC_ROWS, 128), lambda i: (i, 0))],
        out_specs=pl.BlockSpec((1, 1), lambda i: (0, 0)),
    )(pp2)


def kernel(cent_word, pos_word, neg_word, U, V):
    cent_idx = cent_word.astype(jnp.int32).reshape(B // CB, CB)
    pos_idx = pos_word.astype(jnp.int32).reshape(B // CB, CB)
    neg_idx = neg_word.astype(jnp.int32).reshape(B * NEG // NIW, NIW)
    pp = _sc_partial_dots(cent_idx, pos_idx, neg_idx, U, V)
    return _tc_loss(pp).reshape(())
